# baseline (speedup 1.0000x reference)
import functools

import jax
import jax.numpy as jnp
from jax.experimental import pallas as pl

_C = 256
_HID = 1024


def _mlp_kernel(x_ref, w1_ref, b1_ref, w2_ref, b2_ref, o_ref):
    _, c, hb, w = x_ref.shape
    x = x_ref[0].reshape(c, hb * w)
    h = jax.lax.dot_general(
        w1_ref[...], x, (((0,), (0,)), ((), ())),
        preferred_element_type=jnp.float32,
    )
    h = jnp.maximum(h + b1_ref[...], 0.0)
    o = jax.lax.dot_general(
        w2_ref[...], h, (((0,), (0,)), ((), ())),
        preferred_element_type=jnp.float32,
    )
    o_ref[0] = (o + b2_ref[...]).reshape(c, hb, w)


@functools.partial(jax.jit, static_argnames=("block_h", "interpret"))
def _mlp_map(feat, w1, b1c, w2, b2c, *, block_h, interpret=False):
    b, c, h, w = feat.shape
    bh = min(block_h, h)
    grid = (b, h // bh)
    return pl.pallas_call(
        _mlp_kernel,
        grid=grid,
        in_specs=[
            pl.BlockSpec((1, c, bh, w), lambda i, j: (i, 0, j, 0)),
            pl.BlockSpec((_C, _HID), lambda i, j: (0, 0)),
            pl.BlockSpec((_HID, 1), lambda i, j: (0, 0)),
            pl.BlockSpec((_HID, _C), lambda i, j: (0, 0)),
            pl.BlockSpec((_C, 1), lambda i, j: (0, 0)),
        ],
        out_specs=pl.BlockSpec((1, c, bh, w), lambda i, j: (i, 0, j, 0)),
        out_shape=jax.ShapeDtypeStruct((b, c, h, w), jnp.float32),
        interpret=interpret,
    )(feat, w1, b1c, w2, b2c)


def kernel(feat_map0, feat_map1, feat_map2, feat_map3, W1, b1, W2, b2):
    b1c = b1.reshape(_HID, 1)
    b2c = b2.reshape(_C, 1)
    out0 = _mlp_map(feat_map0, W1, b1c, W2, b2c, block_h=16)
    out1 = _mlp_map(feat_map1, W1, b1c, W2, b2c, block_h=32)
    return (out0, out1, feat_map2, feat_map3)


# speedup vs baseline: 1.6882x; 1.6882x over previous
import functools

import jax
import jax.numpy as jnp
from jax.experimental import pallas as pl

_C = 256
_HID = 1024


def _mlp_kernel(x_ref, w1_ref, b1_ref, w2_ref, b2_ref, o_ref):
    x = x_ref[0].astype(jnp.bfloat16)
    h = jax.lax.dot_general(
        w1_ref[...], x, (((0,), (0,)), ((), ())),
        preferred_element_type=jnp.float32,
    )
    h = jnp.maximum(h + b1_ref[...], 0.0).astype(jnp.bfloat16)
    o = jax.lax.dot_general(
        w2_ref[...], h, (((0,), (0,)), ((), ())),
        preferred_element_type=jnp.float32,
    )
    o_ref[0] = o + b2_ref[...]


@functools.partial(jax.jit, static_argnames=("block_n", "interpret"))
def _mlp_map(feat, w1, b1c, w2, b2c, *, block_n, interpret=False):
    b, c, h, w = feat.shape
    n = h * w
    x = feat.reshape(b, c, n)
    bn = min(block_n, n)
    grid = (b, n // bn)
    out = pl.pallas_call(
        _mlp_kernel,
        grid=grid,
        in_specs=[
            pl.BlockSpec((1, c, bn), lambda i, j: (i, 0, j)),
            pl.BlockSpec((_C, _HID), lambda i, j: (0, 0)),
            pl.BlockSpec((_HID, 1), lambda i, j: (0, 0)),
            pl.BlockSpec((_HID, _C), lambda i, j: (0, 0)),
            pl.BlockSpec((_C, 1), lambda i, j: (0, 0)),
        ],
        out_specs=pl.BlockSpec((1, c, bn), lambda i, j: (i, 0, j)),
        out_shape=jax.ShapeDtypeStruct((b, c, n), jnp.float32),
        interpret=interpret,
    )(x, w1, b1c, w2, b2c)
    return out.reshape(b, c, h, w)


def kernel(feat_map0, feat_map1, feat_map2, feat_map3, W1, b1, W2, b2):
    w1h = W1.astype(jnp.bfloat16)
    w2h = W2.astype(jnp.bfloat16)
    b1c = b1.reshape(_HID, 1)
    b2c = b2.reshape(_C, 1)
    out0 = _mlp_map(feat_map0, w1h, b1c, w2h, b2c, block_n=1024)
    out1 = _mlp_map(feat_map1, w1h, b1c, w2h, b2c, block_n=1024)
    return (out0, out1, feat_map2, feat_map3)


# speedup vs baseline: 3.8450x; 2.2775x over previous
import functools

import jax
import jax.numpy as jnp
from jax.experimental import pallas as pl

_C = 256
_HID = 1024


def _mlp_kernel(x_ref, w1_ref, b1_ref, w2_ref, b2_ref, o_ref):
    x = x_ref[...].astype(jnp.bfloat16)
    h = jax.lax.dot_general(
        x, w1_ref[...], (((1,), (0,)), ((), ())),
        preferred_element_type=jnp.float32,
    )
    h = jnp.maximum(h + b1_ref[...], 0.0).astype(jnp.bfloat16)
    o = jax.lax.dot_general(
        h, w2_ref[...], (((1,), (0,)), ((), ())),
        preferred_element_type=jnp.float32,
    )
    o_ref[...] = o + b2_ref[...]


@functools.partial(jax.jit, static_argnames=("block_t", "interpret"))
def _mlp_tokens(x, w1, b1r, w2, b2r, *, block_t, interpret=False):
    n, c = x.shape
    bt = min(block_t, n)
    grid = (n // bt,)
    return pl.pallas_call(
        _mlp_kernel,
        grid=grid,
        in_specs=[
            pl.BlockSpec((bt, c), lambda j: (j, 0)),
            pl.BlockSpec((_C, _HID), lambda j: (0, 0)),
            pl.BlockSpec((1, _HID), lambda j: (0, 0)),
            pl.BlockSpec((_HID, _C), lambda j: (0, 0)),
            pl.BlockSpec((1, _C), lambda j: (0, 0)),
        ],
        out_specs=pl.BlockSpec((bt, c), lambda j: (j, 0)),
        out_shape=jax.ShapeDtypeStruct((n, c), jnp.float32),
        interpret=interpret,
    )(x, w1, b1r, w2, b2r)


def _run_map(feat, w1h, b1r, w2h, b2r, block_t):
    b, c, h, w = feat.shape
    xt = feat.transpose(0, 2, 3, 1).reshape(-1, c)
    yt = _mlp_tokens(xt, w1h, b1r, w2h, b2r, block_t=block_t)
    return yt.reshape(b, h, w, c).transpose(0, 3, 1, 2)


def kernel(feat_map0, feat_map1, feat_map2, feat_map3, W1, b1, W2, b2):
    w1h = W1.astype(jnp.bfloat16)
    w2h = W2.astype(jnp.bfloat16)
    b1r = b1.reshape(1, _HID)
    b2r = b2.reshape(1, _C)
    out0 = _run_map(feat_map0, w1h, b1r, w2h, b2r, block_t=2048)
    out1 = _run_map(feat_map1, w1h, b1r, w2h, b2r, block_t=2048)
    return (out0, out1, feat_map2, feat_map3)


# speedup vs baseline: 4.3141x; 1.1220x over previous
import functools

import jax
import jax.numpy as jnp
from jax.experimental import pallas as pl

_C = 256
_HID = 1024


def _mlp_block(x_ref, w1, b1_ref, w2, b2_ref, o_ref):
    x = x_ref[...].astype(jnp.bfloat16)
    h = jax.lax.dot_general(
        x, w1, (((1,), (0,)), ((), ())),
        preferred_element_type=jnp.float32,
    )
    h = jnp.maximum(h + b1_ref[...], 0.0).astype(jnp.bfloat16)
    o = jax.lax.dot_general(
        h, w2, (((1,), (0,)), ((), ())),
        preferred_element_type=jnp.float32,
    )
    o_ref[...] = o + b2_ref[...]


def _make_kernel(nblk0):
    def _kernel(x0_ref, x1_ref, w1_ref, b1_ref, w2_ref, b2_ref,
                o0_ref, o1_ref):
        j = pl.program_id(0)
        w1 = w1_ref[...].astype(jnp.bfloat16)
        w2 = w2_ref[...].astype(jnp.bfloat16)

        @pl.when(j < nblk0)
        def _():
            _mlp_block(x0_ref, w1, b1_ref, w2, b2_ref, o0_ref)

        @pl.when(j >= nblk0)
        def _():
            _mlp_block(x1_ref, w1, b1_ref, w2, b2_ref, o1_ref)

    return _kernel


@functools.partial(jax.jit, static_argnames=("block_t", "interpret"))
def _mlp_two(x0, x1, w1, b1r, w2, b2r, *, block_t, interpret=False):
    n0, c = x0.shape
    n1, _ = x1.shape
    bt = block_t
    nblk0, nblk1 = n0 // bt, n1 // bt
    grid = (nblk0 + nblk1,)
    return pl.pallas_call(
        _make_kernel(nblk0),
        grid=grid,
        in_specs=[
            pl.BlockSpec((bt, c), lambda j: (jnp.minimum(j, nblk0 - 1), 0)),
            pl.BlockSpec((bt, c), lambda j: (jnp.maximum(j - nblk0, 0), 0)),
            pl.BlockSpec((_C, _HID), lambda j: (0, 0)),
            pl.BlockSpec((1, _HID), lambda j: (0, 0)),
            pl.BlockSpec((_HID, _C), lambda j: (0, 0)),
            pl.BlockSpec((1, _C), lambda j: (0, 0)),
        ],
        out_specs=[
            pl.BlockSpec((bt, c), lambda j: (jnp.minimum(j, nblk0 - 1), 0)),
            pl.BlockSpec((bt, c), lambda j: (jnp.maximum(j - nblk0, 0), 0)),
        ],
        out_shape=[
            jax.ShapeDtypeStruct((n0, c), jnp.float32),
            jax.ShapeDtypeStruct((n1, c), jnp.float32),
        ],
        interpret=interpret,
    )(x0, x1, w1, b1r, w2, b2r)


def kernel(feat_map0, feat_map1, feat_map2, feat_map3, W1, b1, W2, b2):
    b, c, h0, w0 = feat_map0.shape
    _, _, h1, w1sz = feat_map1.shape
    xt0 = feat_map0.transpose(0, 2, 3, 1).reshape(-1, c)
    xt1 = feat_map1.transpose(0, 2, 3, 1).reshape(-1, c)
    y0, y1 = _mlp_two(xt0, xt1, W1, b1.reshape(1, _HID),
                      W2, b2.reshape(1, _C), block_t=2048)
    out0 = y0.reshape(b, h0, w0, c).transpose(0, 3, 1, 2)
    out1 = y1.reshape(b, h1, w1sz, c).transpose(0, 3, 1, 2)
    return (out0, out1, feat_map2, feat_map3)
